# baseline (speedup 1.0000x reference)
import functools

import jax
import jax.numpy as jnp
from jax import lax
from jax.experimental import pallas as pl
from jax.experimental.pallas import tpu as pltpu
from jax.experimental.pallas import tpu_sc as plsc

N = 10000
E = 160000
D = 128
R = 16
NB = 8
NPAD = 10240
EPAD = 163840
NW = 32
ECHUNK = 64
NCHUNKS = EPAD // (NW * ECHUNK)
EPW = EPAD // NW
NBUF = 4
RPT = NPAD // 16
BLK = 128


def _weights(att, basis):
    def body(att_ref, basis_ref, w_ref):
        for r in range(R):
            acc = att_ref[r, 0] * basis_ref[0]
            for b in range(1, NB):
                acc = acc + att_ref[r, b] * basis_ref[b]
            w_ref[r] = acc
    return pl.pallas_call(
        body,
        in_specs=[pl.BlockSpec(memory_space=pltpu.SMEM),
                  pl.BlockSpec(memory_space=pltpu.VMEM)],
        out_shape=jax.ShapeDtypeStruct((R, D, D), jnp.float32),
    )(att, basis)


def _project(h, w):
    def body(h_ref, w_ref, y_ref):
        hblk = h_ref[...]
        for r in range(R):
            y_ref[r] = jnp.dot(hblk, w_ref[r],
                               preferred_element_type=jnp.float32)
    return pl.pallas_call(
        body,
        grid=(NPAD // BLK,),
        in_specs=[pl.BlockSpec((BLK, D), lambda i: (i, 0)),
                  pl.BlockSpec((R, D, D), lambda i: (0, 0, 0))],
        out_specs=pl.BlockSpec((R, BLK, D), lambda i: (0, i, 0)),
        out_shape=jax.ShapeDtypeStruct((R, NPAD, D), jnp.float32),
    )(h, w)


def _cnt_recip(cnt_hist):
    def body(c_ref, o_ref):
        o_ref[...] = 1.0 / (jnp.sum(c_ref[...], axis=0)[:, None] + 1.0)
    return pl.pallas_call(
        body,
        grid=(NPAD // BLK,),
        in_specs=[pl.BlockSpec((NW, BLK), lambda i: (0, i))],
        out_specs=pl.BlockSpec((BLK, 1), lambda i: (i, 0)),
        out_shape=jax.ShapeDtypeStruct((NPAD, 1), jnp.float32),
    )(cnt_hist)


def _combine(h, s, rcnt, wself, root, bias, lns, lnb):
    def body(h_ref, s_ref, r_ref, wself_ref, root_ref, b_ref, g_ref, lb_ref,
             o_ref):
        hblk = h_ref[...]
        feat = s_ref[0] + s_ref[1]
        selfm = jnp.dot(hblk, wself_ref[...], preferred_element_type=jnp.float32)
        agg = (feat + selfm) * r_ref[...]
        out = agg + jnp.dot(hblk, root_ref[...],
                            preferred_element_type=jnp.float32) + b_ref[...]
        t = hblk + out
        mu = jnp.mean(t, axis=-1, keepdims=True)
        var = jnp.mean((t - mu) ** 2, axis=-1, keepdims=True)
        o_ref[...] = (t - mu) / jnp.sqrt(var + 1e-5) * g_ref[...] + lb_ref[...]
    return pl.pallas_call(
        body,
        grid=(NPAD // BLK,),
        in_specs=[pl.BlockSpec((BLK, D), lambda i: (i, 0)),
                  pl.BlockSpec((2, BLK, D), lambda i: (0, i, 0)),
                  pl.BlockSpec((BLK, 1), lambda i: (i, 0)),
                  pl.BlockSpec((D, D), lambda i: (0, 0)),
                  pl.BlockSpec((D, D), lambda i: (0, 0)),
                  pl.BlockSpec((1, D), lambda i: (0, 0)),
                  pl.BlockSpec((1, D), lambda i: (0, 0)),
                  pl.BlockSpec((1, D), lambda i: (0, 0))],
        out_specs=pl.BlockSpec((BLK, D), lambda i: (i, 0)),
        out_shape=jax.ShapeDtypeStruct((NPAD, D), jnp.float32),
    )(h, s, rcnt, wself, root,
      bias.reshape(1, D), lns.reshape(1, D), lnb.reshape(1, D))


def _build_sc_kernel(with_cnt):
    mesh = plsc.VectorSubcoreMesh(core_axis_name="c", subcore_axis_name="s")
    out_type = [jax.ShapeDtypeStruct((2, NPAD, D), jnp.float32)]
    if with_cnt:
        out_type.append(jax.ShapeDtypeStruct((NW, NPAD), jnp.float32))
    scratch = (
        [pltpu.VMEM((ECHUNK,), jnp.int32) for _ in range(NBUF)]
        + [pltpu.VMEM((ECHUNK,), jnp.float32) for _ in range(NBUF)]
        + [pltpu.VMEM((1, ECHUNK), jnp.int32) for _ in range(NBUF)]
        + [pltpu.VMEM((ECHUNK, D), jnp.float32) for _ in range(NBUF)]
        + ([pltpu.VMEM((NPAD + 16,), jnp.float32)] if with_cnt else [])
        + [pltpu.VMEM_SHARED((NPAD, D), jnp.float32),
           pltpu.SemaphoreType.DMA,
           pltpu.SemaphoreType.DMA,
           pltpu.SemaphoreType.DMA]
    )

    @functools.partial(pl.kernel, out_type=out_type, mesh=mesh,
                       scratch_types=scratch)
    def k(y_h, gidx_h, ea_h, dst_h, out_h, *refs):
        if with_cnt:
            cnt_h = refs[0]
            refs = refs[1:]
        gidx_b = refs[0:NBUF]
        ea_b = refs[NBUF:2 * NBUF]
        dst_b = refs[2 * NBUF:3 * NBUF]
        rows = refs[3 * NBUF:4 * NBUF]
        if with_cnt:
            cnt_l = refs[4 * NBUF]
            refs = refs[:4 * NBUF] + refs[4 * NBUF + 1:]
        s_sh, gsem, msem, ssem = refs[4 * NBUF:]
        cid = lax.axis_index("c")
        sid = lax.axis_index("s")
        wid = sid * 2 + cid
        zeros16 = jnp.zeros((16,), jnp.float32)
        onehot0 = jnp.where(lax.iota(jnp.int32, 16) == 0,
                            jnp.float32(1.0), jnp.float32(0.0))

        def meta_start(c, u):
            pltpu.async_copy(gidx_h.at[wid, c], gidx_b[u], msem)
            pltpu.async_copy(ea_h.at[wid, c], ea_b[u], msem)
            pltpu.async_copy(dst_h.at[wid, pl.ds(c, 1)], dst_b[u], msem)

        def meta_wait(c, u):
            pltpu.make_async_copy(gidx_h.at[wid, c], gidx_b[u], msem).wait()
            pltpu.make_async_copy(ea_h.at[wid, c], ea_b[u], msem).wait()
            pltpu.make_async_copy(dst_h.at[wid, pl.ds(c, 1)], dst_b[u],
                                  msem).wait()

        def gather_start(c, u):
            pltpu.async_copy(y_h.at[gidx_b[u]], rows[u], gsem)

        def gather_wait(c, u):
            pltpu.make_async_copy(y_h.at[gidx_b[u]], rows[u], gsem).wait()

        def scat_start(u):
            pltpu.async_copy(rows[u], s_sh.at[dst_b[u].at[0]], ssem, add=True)

        def scat_wait(u):
            pltpu.make_async_copy(rows[u], s_sh.at[dst_b[u].at[0]],
                                  ssem).wait()

        if with_cnt:
            def zhist(i, _):
                cnt_l[pl.ds(i * 16, 16)] = zeros16
                return 0
            lax.fori_loop(0, (NPAD + 16) // 16, zhist, 0)

        def zrow(i, _):
            for j in range(D // 16):
                rows[0][i, pl.ds(j * 16, 16)] = zeros16
            return 0
        lax.fori_loop(0, ECHUNK, zrow, 0)

        def zcopy(kk, _):
            pltpu.sync_copy(rows[0],
                            s_sh.at[pl.ds(sid * RPT + kk * ECHUNK, ECHUNK)])
            return 0
        lax.fori_loop(0, RPT // ECHUNK, zcopy, 0)
        plsc.subcore_barrier()

        meta_start(0, 0)
        meta_start(1, 1)
        meta_start(2, 2)
        meta_wait(0, 0)
        gather_start(0, 0)
        meta_wait(1, 1)
        gather_start(1, 1)

        def scale(c, u):
            def grp(g, _):
                mvec = ea_b[u][pl.ds(g * 16, 16)]
                if with_cnt:
                    dvec = dst_b[u][0, pl.ds(g * 16, 16)]
                for i in range(16):
                    e = g * 16 + i
                    m = mvec[i]
                    for j in range(D // 16):
                        sl = pl.ds(j * 16, 16)
                        rows[u][e, sl] = rows[u][e, sl] * m
                    if with_cnt:
                        d = dvec[i]
                        cnt_l[pl.ds(d, 16)] = cnt_l[pl.ds(d, 16)] + onehot0
                return 0
            lax.fori_loop(0, ECHUNK // 16, grp, 0)

        def ring(t, _):
            for u in range(NBUF):
                c = NBUF * t + u
                u2 = (u + 2) % NBUF
                u3 = (u + 3) % NBUF

                @pl.when(c + 2 < NCHUNKS)
                def _():
                    meta_wait(c + 2, u2)
                    gather_start(c + 2, u2)
                gather_wait(c, u)
                scale(c, u)
                scat_start(u)

                @pl.when(c >= 1)
                def _():
                    scat_wait(u3)

                @pl.when(c + 3 < NCHUNKS)
                def _():
                    meta_start(c + 3, u3)
            return 0
        lax.fori_loop(0, NCHUNKS // NBUF, ring, 0)
        scat_wait((NCHUNKS - 1) % NBUF)
        plsc.subcore_barrier()

        def wb(kk, _):
            off = sid * RPT + kk * ECHUNK
            pltpu.sync_copy(s_sh.at[pl.ds(off, ECHUNK)], rows[0])
            pltpu.sync_copy(rows[0], out_h.at[cid, pl.ds(off, ECHUNK)])
            return 0
        lax.fori_loop(0, RPT // ECHUNK, wb, 0)
        if with_cnt:
            pltpu.sync_copy(cnt_l.at[pl.ds(0, NPAD)], cnt_h.at[wid])

    return k


_SC_AGGREGATE0 = _build_sc_kernel(True)
_SC_AGGREGATE1 = _build_sc_kernel(False)


def kernel(x, edge_index, edge_type, edge_attr,
           basis0, att0, root0, bias0, ln_scale0, ln_bias0,
           basis1, att1, root1, bias1, ln_scale1, ln_bias1):
    src = edge_index[0].astype(jnp.int32)
    dst = edge_index[1].astype(jnp.int32)
    et = edge_type.astype(jnp.int32)
    ea = edge_attr.astype(jnp.float32)
    pad = EPAD - E
    gidx3 = jnp.pad(et * NPAD + src, (0, pad)).reshape(NW, NCHUNKS, ECHUNK)
    ea3 = jnp.pad(ea, (0, pad)).reshape(NW, NCHUNKS, ECHUNK)
    dst3 = jnp.pad(dst, (0, pad), constant_values=NPAD - 1).reshape(
        NW, NCHUNKS, ECHUNK)
    h = jnp.pad(x.astype(jnp.float32), ((0, NPAD - N), (0, 0)))

    layers = ((basis0, att0, root0, bias0, ln_scale0, ln_bias0),
              (basis1, att1, root1, bias1, ln_scale1, ln_bias1))
    rcnt = None
    for li, (basis, att, root, bias, lns, lnb) in enumerate(layers):
        w = _weights(att.astype(jnp.float32), basis.astype(jnp.float32))
        y = _project(h, w)
        if li == 0:
            s, cnt_hist = _SC_AGGREGATE0(y.reshape(R * NPAD, D),
                                         gidx3, ea3, dst3)
            rcnt = _cnt_recip(cnt_hist)
        else:
            (s,) = _SC_AGGREGATE1(y.reshape(R * NPAD, D), gidx3, ea3, dst3)
        h = _combine(h, s, rcnt, w[R - 1], root.astype(jnp.float32),
                     bias.astype(jnp.float32), lns.astype(jnp.float32),
                     lnb.astype(jnp.float32))
    return h[:N]


# speedup vs baseline: 1.0262x; 1.0262x over previous
import functools

import jax
import jax.numpy as jnp
from jax import lax
from jax.experimental import pallas as pl
from jax.experimental.pallas import tpu as pltpu
from jax.experimental.pallas import tpu_sc as plsc

N = 10000
E = 160000
D = 128
R = 16
NB = 8
NPAD = 10240
EPAD = 163840
NW = 32
ECHUNK = 64
NCHUNKS = EPAD // (NW * ECHUNK)
EPW = EPAD // NW
NBUF = 4
RPT = NPAD // 16
BLK = 128


def _weights(att, basis):
    def body(att_ref, basis_ref, w_ref):
        for r in range(R):
            acc = att_ref[r, 0] * basis_ref[0]
            for b in range(1, NB):
                acc = acc + att_ref[r, b] * basis_ref[b]
            w_ref[r] = acc
    return pl.pallas_call(
        body,
        in_specs=[pl.BlockSpec(memory_space=pltpu.SMEM),
                  pl.BlockSpec(memory_space=pltpu.VMEM)],
        out_shape=jax.ShapeDtypeStruct((R, D, D), jnp.float32),
    )(att, basis)


def _project(h, w):
    def body(h_ref, w_ref, y_ref):
        hblk = h_ref[...]
        for r in range(R):
            y_ref[r] = jnp.dot(hblk, w_ref[r],
                               preferred_element_type=jnp.float32)
    return pl.pallas_call(
        body,
        grid=(NPAD // BLK,),
        in_specs=[pl.BlockSpec((BLK, D), lambda i: (i, 0)),
                  pl.BlockSpec((R, D, D), lambda i: (0, 0, 0))],
        out_specs=pl.BlockSpec((R, BLK, D), lambda i: (0, i, 0)),
        out_shape=jax.ShapeDtypeStruct((R, NPAD, D), jnp.float32),
    )(h, w)


def _combine(h, s, cnt_hist, wself, root, bias, lns, lnb):
    def body(h_ref, s_ref, c_ref, wself_ref, root_ref, b_ref, g_ref, lb_ref,
             o_ref):
        hblk = h_ref[...]
        feat = s_ref[0] + s_ref[1]
        cnt = jnp.sum(c_ref[...], axis=0)[:, None] + 1.0
        selfm = jnp.dot(hblk, wself_ref[...], preferred_element_type=jnp.float32)
        agg = (feat + selfm) / jnp.maximum(cnt, 1.0)
        out = agg + jnp.dot(hblk, root_ref[...],
                            preferred_element_type=jnp.float32) + b_ref[...]
        t = hblk + out
        mu = jnp.mean(t, axis=-1, keepdims=True)
        var = jnp.mean((t - mu) ** 2, axis=-1, keepdims=True)
        o_ref[...] = (t - mu) / jnp.sqrt(var + 1e-5) * g_ref[...] + lb_ref[...]
    return pl.pallas_call(
        body,
        grid=(NPAD // BLK,),
        in_specs=[pl.BlockSpec((BLK, D), lambda i: (i, 0)),
                  pl.BlockSpec((2, BLK, D), lambda i: (0, i, 0)),
                  pl.BlockSpec((NW, BLK), lambda i: (0, i)),
                  pl.BlockSpec((D, D), lambda i: (0, 0)),
                  pl.BlockSpec((D, D), lambda i: (0, 0)),
                  pl.BlockSpec((1, D), lambda i: (0, 0)),
                  pl.BlockSpec((1, D), lambda i: (0, 0)),
                  pl.BlockSpec((1, D), lambda i: (0, 0))],
        out_specs=pl.BlockSpec((BLK, D), lambda i: (i, 0)),
        out_shape=jax.ShapeDtypeStruct((NPAD, D), jnp.float32),
    )(h, s, cnt_hist, wself, root,
      bias.reshape(1, D), lns.reshape(1, D), lnb.reshape(1, D))


def _build_sc_kernel(with_cnt):
    mesh = plsc.VectorSubcoreMesh(core_axis_name="c", subcore_axis_name="s")
    out_type = [jax.ShapeDtypeStruct((2, NPAD, D), jnp.float32)]
    if with_cnt:
        out_type.append(jax.ShapeDtypeStruct((NW, NPAD), jnp.float32))
    scratch = (
        [pltpu.VMEM((ECHUNK,), jnp.int32) for _ in range(NBUF)]
        + [pltpu.VMEM((ECHUNK,), jnp.float32) for _ in range(NBUF)]
        + [pltpu.VMEM((1, ECHUNK), jnp.int32) for _ in range(NBUF)]
        + [pltpu.VMEM((ECHUNK, D), jnp.float32) for _ in range(NBUF)]
        + ([pltpu.VMEM((NPAD + 16,), jnp.float32)] if with_cnt else [])
        + [pltpu.VMEM_SHARED((NPAD, D), jnp.float32),
           pltpu.SemaphoreType.DMA,
           pltpu.SemaphoreType.DMA,
           pltpu.SemaphoreType.DMA]
    )

    @functools.partial(pl.kernel, out_type=out_type, mesh=mesh,
                       scratch_types=scratch)
    def k(y_h, gidx_h, ea_h, dst_h, out_h, *refs):
        if with_cnt:
            cnt_h = refs[0]
            refs = refs[1:]
        gidx_b = refs[0:NBUF]
        ea_b = refs[NBUF:2 * NBUF]
        dst_b = refs[2 * NBUF:3 * NBUF]
        rows = refs[3 * NBUF:4 * NBUF]
        if with_cnt:
            cnt_l = refs[4 * NBUF]
            refs = refs[:4 * NBUF] + refs[4 * NBUF + 1:]
        s_sh, gsem, msem, ssem = refs[4 * NBUF:]
        cid = lax.axis_index("c")
        sid = lax.axis_index("s")
        wid = sid * 2 + cid
        zeros16 = jnp.zeros((16,), jnp.float32)
        onehot0 = jnp.where(lax.iota(jnp.int32, 16) == 0,
                            jnp.float32(1.0), jnp.float32(0.0))

        def meta_start(c, u):
            pltpu.async_copy(gidx_h.at[wid, c], gidx_b[u], msem)
            pltpu.async_copy(ea_h.at[wid, c], ea_b[u], msem)
            pltpu.async_copy(dst_h.at[wid, pl.ds(c, 1)], dst_b[u], msem)

        def meta_wait(c, u):
            pltpu.make_async_copy(gidx_h.at[wid, c], gidx_b[u], msem).wait()
            pltpu.make_async_copy(ea_h.at[wid, c], ea_b[u], msem).wait()
            pltpu.make_async_copy(dst_h.at[wid, pl.ds(c, 1)], dst_b[u],
                                  msem).wait()

        def gather_start(c, u):
            pltpu.async_copy(y_h.at[gidx_b[u]], rows[u], gsem)

        def gather_wait(c, u):
            pltpu.make_async_copy(y_h.at[gidx_b[u]], rows[u], gsem).wait()

        def scat_start(u):
            pltpu.async_copy(rows[u], s_sh.at[dst_b[u].at[0]], ssem, add=True)

        def scat_wait(u):
            pltpu.make_async_copy(rows[u], s_sh.at[dst_b[u].at[0]],
                                  ssem).wait()

        if with_cnt:
            def zhist(i, _):
                cnt_l[pl.ds(i * 16, 16)] = zeros16
                return 0
            lax.fori_loop(0, (NPAD + 16) // 16, zhist, 0)

        def zrow(i, _):
            for j in range(D // 16):
                rows[0][i, pl.ds(j * 16, 16)] = zeros16
            return 0
        lax.fori_loop(0, ECHUNK, zrow, 0)

        def zcopy(kk, _):
            pltpu.sync_copy(rows[0],
                            s_sh.at[pl.ds(sid * RPT + kk * ECHUNK, ECHUNK)])
            return 0
        lax.fori_loop(0, RPT // ECHUNK, zcopy, 0)
        plsc.subcore_barrier()

        meta_start(0, 0)
        meta_start(1, 1)
        meta_start(2, 2)
        meta_wait(0, 0)
        gather_start(0, 0)
        meta_wait(1, 1)
        gather_start(1, 1)

        def scale(c, u):
            def grp(g, _):
                mvec = ea_b[u][pl.ds(g * 16, 16)]
                if with_cnt:
                    dvec = dst_b[u][0, pl.ds(g * 16, 16)]
                for i in range(16):
                    e = g * 16 + i
                    m = mvec[i]
                    for j in range(D // 16):
                        sl = pl.ds(j * 16, 16)
                        rows[u][e, sl] = rows[u][e, sl] * m
                    if with_cnt:
                        d = dvec[i]
                        cnt_l[pl.ds(d, 16)] = cnt_l[pl.ds(d, 16)] + onehot0
                return 0
            lax.fori_loop(0, ECHUNK // 16, grp, 0)

        def ring(t, _):
            for u in range(NBUF):
                c = NBUF * t + u
                u2 = (u + 2) % NBUF
                u3 = (u + 3) % NBUF

                @pl.when(c + 2 < NCHUNKS)
                def _():
                    meta_wait(c + 2, u2)
                    gather_start(c + 2, u2)
                gather_wait(c, u)
                scale(c, u)
                scat_start(u)

                @pl.when(c >= 1)
                def _():
                    scat_wait(u3)

                @pl.when(c + 3 < NCHUNKS)
                def _():
                    meta_start(c + 3, u3)
            return 0
        lax.fori_loop(0, NCHUNKS // NBUF, ring, 0)
        scat_wait((NCHUNKS - 1) % NBUF)
        plsc.subcore_barrier()

        def wb(kk, _):
            off = sid * RPT + kk * ECHUNK
            pltpu.sync_copy(s_sh.at[pl.ds(off, ECHUNK)], rows[0])
            pltpu.sync_copy(rows[0], out_h.at[cid, pl.ds(off, ECHUNK)])
            return 0
        lax.fori_loop(0, RPT // ECHUNK, wb, 0)
        if with_cnt:
            pltpu.sync_copy(cnt_l.at[pl.ds(0, NPAD)], cnt_h.at[wid])

    return k


_SC_AGGREGATE0 = _build_sc_kernel(True)
_SC_AGGREGATE1 = _build_sc_kernel(False)


def kernel(x, edge_index, edge_type, edge_attr,
           basis0, att0, root0, bias0, ln_scale0, ln_bias0,
           basis1, att1, root1, bias1, ln_scale1, ln_bias1):
    src = edge_index[0].astype(jnp.int32)
    dst = edge_index[1].astype(jnp.int32)
    et = edge_type.astype(jnp.int32)
    ea = edge_attr.astype(jnp.float32)
    pad = EPAD - E
    gidx3 = jnp.pad(et * NPAD + src, (0, pad)).reshape(NW, NCHUNKS, ECHUNK)
    ea3 = jnp.pad(ea, (0, pad)).reshape(NW, NCHUNKS, ECHUNK)
    dst3 = jnp.pad(dst, (0, pad), constant_values=NPAD - 1).reshape(
        NW, NCHUNKS, ECHUNK)
    h = jnp.pad(x.astype(jnp.float32), ((0, NPAD - N), (0, 0)))

    layers = ((basis0, att0, root0, bias0, ln_scale0, ln_bias0),
              (basis1, att1, root1, bias1, ln_scale1, ln_bias1))
    cnt_hist = None
    for li, (basis, att, root, bias, lns, lnb) in enumerate(layers):
        w = _weights(att.astype(jnp.float32), basis.astype(jnp.float32))
        y = _project(h, w)
        if li == 0:
            s, cnt_hist = _SC_AGGREGATE0(y.reshape(R * NPAD, D),
                                         gidx3, ea3, dst3)
        else:
            (s,) = _SC_AGGREGATE1(y.reshape(R * NPAD, D), gidx3, ea3, dst3)
        h = _combine(h, s, cnt_hist, w[R - 1], root.astype(jnp.float32),
                     bias.astype(jnp.float32), lns.astype(jnp.float32),
                     lnb.astype(jnp.float32))
    return h[:N]
